# baseline (speedup 1.0000x reference)
import functools

import jax
import jax.numpy as jnp
from jax.experimental import pallas as pl
from jax.experimental.pallas import tpu as pltpu

N = 10000
K = 16
IN_DIM = 256
OUT_DIM = 256
BN = 1000
NB = N // BN
BNO = 5000
NBO = N // BNO
EPS = 1e-5


def _body(self_ref, neigh_ref, ws_ref, wn_ref, b_ref, gamma_ref,
          beta_ref, out_ref, y_ref, acc_ref):
    i = pl.program_id(0)

    @pl.when(i == 0)
    def _init():
        acc_ref[...] = jnp.zeros_like(acc_ref)

    @pl.when(i < NB)
    def _compute():
        agg = jnp.sum(neigh_ref[...], axis=1)
        y = (
            jnp.dot(self_ref[...], ws_ref[...], preferred_element_type=jnp.float32)
            + jnp.dot(agg, wn_ref[...], preferred_element_type=jnp.float32)
            + b_ref[...]
        )
        y_ref[pl.ds(i * BN, BN), :] = y
        acc_ref[0:1, :] += jnp.sum(y, axis=0, keepdims=True)
        acc_ref[1:2, :] += jnp.sum(y * y, axis=0, keepdims=True)

    @pl.when(i >= NB)
    def _normalize():
        j = i - NB
        mean = acc_ref[0:1, :] / N
        var = acc_ref[1:2, :] / N - mean * mean
        scale = gamma_ref[...] * jax.lax.rsqrt(var + EPS)
        shift = beta_ref[...] - mean * scale
        y = y_ref[pl.ds(j * BNO, BNO), :]
        out_ref[...] = jnp.maximum(y * scale + shift, 0.0)


@jax.jit
def kernel(self_feat, neighbor_feat, W, b, gamma, beta):
    ws = W[:, :IN_DIM].T
    wn = W[:, IN_DIM:].T * (1.0 / K)
    b2 = b.reshape(1, OUT_DIM)
    gamma2 = gamma.reshape(1, OUT_DIM)
    beta2 = beta.reshape(1, OUT_DIM)

    grid = (NB + NBO,)
    out = pl.pallas_call(
        _body,
        grid=grid,
        in_specs=[
            pl.BlockSpec((BN, IN_DIM), lambda i: (jnp.minimum(i, NB - 1), 0)),
            pl.BlockSpec((BN, K, IN_DIM), lambda i: (jnp.minimum(i, NB - 1), 0, 0)),
            pl.BlockSpec((IN_DIM, OUT_DIM), lambda i: (0, 0)),
            pl.BlockSpec((IN_DIM, OUT_DIM), lambda i: (0, 0)),
            pl.BlockSpec((1, OUT_DIM), lambda i: (0, 0)),
            pl.BlockSpec((1, OUT_DIM), lambda i: (0, 0)),
            pl.BlockSpec((1, OUT_DIM), lambda i: (0, 0)),
        ],
        out_specs=pl.BlockSpec((BNO, OUT_DIM), lambda i: (jnp.maximum(i - NB, 0), 0)),
        out_shape=jax.ShapeDtypeStruct((N, OUT_DIM), jnp.float32),
        scratch_shapes=[
            pltpu.VMEM((N, OUT_DIM), jnp.float32),
            pltpu.VMEM((2, OUT_DIM), jnp.float32),
        ],
    )(self_feat, neighbor_feat, ws, wn, b2, gamma2, beta2)
    return out
